# baseline (speedup 1.0000x reference)
import functools

import jax
import jax.numpy as jnp
from jax.experimental import pallas as pl
from jax.experimental.pallas import tpu as pltpu

NUM_EXPERTS = 64
TOP_K = 8


def _router_kernel(x_ref, w_ref, vals_ref, idx_ref, counts_ref, aux_ref,
                   cacc, pacc, *, num_tokens):
    i = pl.program_id(0)

    @pl.when(i == 0)
    def _init():
        cacc[...] = jnp.zeros_like(cacc)
        pacc[...] = jnp.zeros_like(pacc)

    x = x_ref[...]
    w = w_ref[...]
    logits = jax.lax.dot_general(
        w, x, (((1,), (1,)), ((), ())), preferred_element_type=jnp.float32
    )

    T = logits.shape[1]
    CT = min(1024, T)
    psum = None
    csum = None
    for c in range(T // CT):
        lg = logits[:, c * CT:(c + 1) * CT]

        m = jnp.max(lg, axis=0, keepdims=True)
        e = jnp.exp(lg - m)
        s = jnp.sum(e, axis=0, keepdims=True)
        probs = e / s

        row = jax.lax.broadcasted_iota(
            jnp.int32, probs.shape, 0).astype(jnp.float32)
        work = probs
        vals = []
        idxs = []
        for k in range(TOP_K):
            if k == 0:
                mk = 1.0 / s
            else:
                mk = jnp.max(work, axis=0, keepdims=True)
            ik = jnp.min(
                jnp.where(work == mk, row, jnp.float32(NUM_EXPERTS)),
                axis=0, keepdims=True,
            )
            vals.append(mk)
            idxs.append(ik)
            work = jnp.where(row == ik, -1.0, work)

        tv = jnp.concatenate(vals, axis=0)
        ti = jnp.concatenate(idxs, axis=0)
        sl = slice(c * CT, (c + 1) * CT)
        vals_ref[:, sl] = tv / jnp.sum(tv, axis=0, keepdims=True)
        idx_ref[:, sl] = ti.astype(jnp.int32)

        pc = jnp.sum(probs, axis=1, keepdims=True)
        cc = jnp.sum(jnp.where(work < 0.0, 1.0, 0.0), axis=1, keepdims=True)
        psum = pc if psum is None else psum + pc
        csum = cc if csum is None else csum + cc

    pacc[...] += psum
    cacc[...] += csum

    @pl.when(i == pl.num_programs(0) - 1)
    def _finish():
        counts = cacc[...]
        counts_ref[...] = counts
        n = jnp.float32(num_tokens)
        aux = jnp.sum((counts / n) * (pacc[...] / n)) * jnp.float32(NUM_EXPERTS)
        aux_ref[...] = aux.reshape(1, 1)


def kernel(hidden_states, gate_weight):
    B, S, d = hidden_states.shape
    n = B * S
    x = hidden_states.reshape(n, d)

    T = 4096
    grid = (n // T,)

    kern = functools.partial(_router_kernel, num_tokens=n)
    vals, idx, counts, aux = pl.pallas_call(
        kern,
        grid=grid,
        in_specs=[
            pl.BlockSpec((T, d), lambda i: (i, 0)),
            pl.BlockSpec((NUM_EXPERTS, d), lambda i: (0, 0)),
        ],
        out_specs=[
            pl.BlockSpec((TOP_K, T), lambda i: (0, i)),
            pl.BlockSpec((TOP_K, T), lambda i: (0, i)),
            pl.BlockSpec((NUM_EXPERTS, 1), lambda i: (0, 0)),
            pl.BlockSpec((1, 1), lambda i: (0, 0)),
        ],
        out_shape=[
            jax.ShapeDtypeStruct((TOP_K, n), jnp.float32),
            jax.ShapeDtypeStruct((TOP_K, n), jnp.int32),
            jax.ShapeDtypeStruct((NUM_EXPERTS, 1), jnp.float32),
            jax.ShapeDtypeStruct((1, 1), jnp.float32),
        ],
        scratch_shapes=[
            pltpu.VMEM((NUM_EXPERTS, 1), jnp.float32),
            pltpu.VMEM((NUM_EXPERTS, 1), jnp.float32),
        ],
    )(x, gate_weight)

    return (vals.T, idx.T.astype(jnp.int64), counts.reshape(NUM_EXPERTS),
            aux.reshape(()))


# speedup vs baseline: 1.0238x; 1.0238x over previous
import functools

import jax
import jax.numpy as jnp
from jax.experimental import pallas as pl
from jax.experimental.pallas import tpu as pltpu

NUM_EXPERTS = 64
TOP_K = 8


def _router_kernel(x_ref, w_ref, vals_ref, idx_ref, counts_ref, aux_ref,
                   cacc, pacc, *, num_tokens):
    i = pl.program_id(0)

    @pl.when(i == 0)
    def _init():
        cacc[...] = jnp.zeros_like(cacc)
        pacc[...] = jnp.zeros_like(pacc)

    x = x_ref[...]
    w = w_ref[...]
    logits = jax.lax.dot_general(
        w, x, (((1,), (1,)), ((), ())), preferred_element_type=jnp.float32
    )

    m = jnp.max(logits, axis=0, keepdims=True)
    e = jnp.exp(logits - m)
    s = jnp.sum(e, axis=0, keepdims=True)
    probs = e / s

    row = jax.lax.broadcasted_iota(jnp.int32, probs.shape, 0).astype(jnp.float32)
    work = probs
    vals = []
    idxs = []
    for k in range(TOP_K):
        if k == 0:
            mk = 1.0 / s
        else:
            mk = jnp.max(work, axis=0, keepdims=True)
        ik = jnp.min(
            jnp.where(work == mk, row, jnp.float32(NUM_EXPERTS)),
            axis=0, keepdims=True,
        )
        vals.append(mk)
        idxs.append(ik)
        work = jnp.where(row == ik, -1.0, work)

    tv = jnp.concatenate(vals, axis=0)
    ti = jnp.concatenate(idxs, axis=0)
    vals_ref[...] = tv / jnp.sum(tv, axis=0, keepdims=True)
    idx_ref[...] = ti.astype(jnp.int32)

    pacc[...] += jnp.sum(probs, axis=1, keepdims=True)
    sel = jnp.where(work < 0.0, 1.0, 0.0)
    cacc[...] += jnp.sum(sel, axis=1, keepdims=True)

    @pl.when(i == pl.num_programs(0) - 1)
    def _finish():
        counts = cacc[...]
        counts_ref[...] = counts
        n = jnp.float32(num_tokens)
        aux = jnp.sum((counts / n) * (pacc[...] / n)) * jnp.float32(NUM_EXPERTS)
        aux_ref[...] = aux.reshape(1, 1)


def kernel(hidden_states, gate_weight):
    B, S, d = hidden_states.shape
    n = B * S
    x = hidden_states.reshape(n, d)

    T = 4096
    grid = (n // T,)

    kern = functools.partial(_router_kernel, num_tokens=n)
    vals, idx, counts, aux = pl.pallas_call(
        kern,
        grid=grid,
        in_specs=[
            pl.BlockSpec((T, d), lambda i: (i, 0)),
            pl.BlockSpec((NUM_EXPERTS, d), lambda i: (0, 0)),
        ],
        out_specs=[
            pl.BlockSpec((TOP_K, T), lambda i: (0, i)),
            pl.BlockSpec((TOP_K, T), lambda i: (0, i)),
            pl.BlockSpec((NUM_EXPERTS, 1), lambda i: (0, 0)),
            pl.BlockSpec((1, 1), lambda i: (0, 0)),
        ],
        out_shape=[
            jax.ShapeDtypeStruct((TOP_K, n), jnp.float32),
            jax.ShapeDtypeStruct((TOP_K, n), jnp.int32),
            jax.ShapeDtypeStruct((NUM_EXPERTS, 1), jnp.float32),
            jax.ShapeDtypeStruct((1, 1), jnp.float32),
        ],
        scratch_shapes=[
            pltpu.VMEM((NUM_EXPERTS, 1), jnp.float32),
            pltpu.VMEM((NUM_EXPERTS, 1), jnp.float32),
        ],
    )(x, gate_weight)

    return (vals.T, idx.T.astype(jnp.int64), counts.reshape(NUM_EXPERTS),
            aux.reshape(()))
